# Initial kernel scaffold; baseline (speedup 1.0000x reference)
import jax
import jax.numpy as jnp
from jax.experimental import pallas as pl


def kernel(topic_prob, hard_label):
    raise NotImplementedError("write your pallas kernel here")


# speedup vs baseline: 386.4197x; 386.4197x over previous
import functools

import jax
import jax.numpy as jnp
from jax import lax
from jax.experimental import pallas as pl
from jax.experimental.pallas import tpu as pltpu
from jax.experimental.pallas import tpu_sc as plsc

_B = 20
_L = 20
_V = 1000
_LP = 32
_NS = 16
_LN2 = 0.6931471805599453


def _clamp(x, lo, hi):
    return jnp.minimum(jnp.maximum(x, lo), hi)


def _ln16(z):
    bits = plsc.bitcast(z, jnp.int32)
    e = lax.shift_right_logical(bits, 23) & 0xFF
    ef = (e - 127).astype(jnp.float32)
    m = plsc.bitcast((bits & 0x7FFFFF) | 0x3F800000, jnp.float32)
    t = (m - 1.0) / (m + 1.0)
    t2 = t * t
    p = jnp.float32(2.0 / 13.0)
    for coef in (2.0 / 11.0, 2.0 / 9.0, 2.0 / 7.0, 2.0 / 5.0, 2.0 / 3.0, 2.0):
        p = p * t2 + jnp.float32(coef)
    return ef * jnp.float32(_LN2) + t * p


@functools.partial(
    pl.kernel,
    out_type=jax.ShapeDtypeStruct((2, 16), jnp.float32),
    mesh=plsc.VectorSubcoreMesh(core_axis_name="c", subcore_axis_name="s"),
    compiler_params=pltpu.CompilerParams(needs_layout_passes=False),
    scratch_types=[
        pltpu.VMEM((_L, _V), jnp.float32),
        pltpu.VMEM((_LP,), jnp.float32),
        pltpu.VMEM((32,), jnp.float32),
        pltpu.VMEM((32,), jnp.float32),
        pltpu.VMEM((32,), jnp.float32),
        pltpu.VMEM((16,), jnp.float32),
        pltpu.VMEM_SHARED((_NS * 16,), jnp.float32),
        pltpu.VMEM((_NS * 16,), jnp.float32),
        pltpu.VMEM((16,), jnp.float32),
    ],
)
def _calcs_sc(tp_hbm, lbl_hbm, out_hbm, slab_v, lbl_v, d0, d1, d2, lv,
              shared, red_v, outv):
    cid = lax.axis_index("c")
    sid = lax.axis_index("s")
    i = cid * _NS + sid
    active = i < _B
    iota = lax.iota(jnp.int32, 16)
    zeros = jnp.zeros((16,), jnp.float32)

    @pl.when(active)
    def _compute():
        pltpu.sync_copy(lbl_hbm.at[i], lbl_v)
        pltpu.sync_copy(tp_hbm.at[i], slab_v)
        for buf in (d0, d1, d2):
            buf[pl.ds(0, 16)] = zeros
            buf[pl.ds(16, 16)] = zeros
        a, b, c = d0, d1, d2
        for sd in range(2, 2 * _L + 1):
            rlo, rhi = max(1, sd - _L), min(_L, sd - 1)
            for h in (0, 1):
                if rhi < 16 * h or rlo > 16 * h + 15:
                    continue
                r = iota + 16 * h
                valid = (r >= rlo) & (r <= rhi)
                kidx = _clamp(sd - r - 1, 0, _LP - 1)
                lblc = plsc.load_gather(lbl_v, [kidx]).astype(jnp.int32)
                col = _clamp(lblc, 0, _V - 1)
                row = _clamp(r - 1, 0, _L - 1)
                p = plsc.load_gather(slab_v, [row, col])
                rm1 = _clamp(r - 1, 0, 31)
                am1 = plsc.load_gather(a, [rm1])
                bm1 = plsc.load_gather(b, [rm1])
                bcur = b[pl.ds(16 * h, 16)]
                nv = p * (am1 + 1.0) + (1.0 - p) * jnp.maximum(bcur, bm1)
                c[pl.ds(16 * h, 16)] = jnp.where(valid, nv, 0.0)
            a, b, c = b, c, a
        dfin = plsc.load_gather(b, [jnp.full((16,), _L, jnp.int32)])
        l0 = lbl_v[pl.ds(0, 16)]
        l1 = lbl_v[pl.ds(16, 16)]
        cntv = (jnp.where(l0 >= 0.0, 1.0, 0.0).astype(jnp.float32)
                + jnp.where(l1 >= 0.0, 1.0, 0.0).astype(jnp.float32))
        cnt = jnp.sum(cntv)
        lnz = _ln16(dfin / cnt)
        lv[...] = lnz * jnp.float32(-1.0 / _B)

    @pl.when(jnp.logical_not(active))
    def _idle():
        lv[...] = zeros

    pltpu.sync_copy(lv, shared.at[pl.ds(sid * 16, 16)])
    plsc.subcore_barrier()

    @pl.when(sid == 0)
    def _reduce():
        pltpu.sync_copy(shared, red_v)
        vals = plsc.load_gather(red_v, [iota * 16])
        tot = jnp.sum(vals)
        outv[...] = lax.broadcast_in_dim(tot, (16,), ())
        pltpu.sync_copy(outv, out_hbm.at[cid])


def kernel(topic_prob, hard_label):
    assert topic_prob.shape == (_B, _L, _V) and hard_label.shape == (_B, _L)
    lblp = jnp.full((_B, _LP), -1.0, jnp.float32).at[:, :_L].set(
        hard_label.astype(jnp.float32))
    out = _calcs_sc(topic_prob.astype(jnp.float32), lblp)
    return out[0, 0] + out[1, 0]
